# Initial kernel scaffold; baseline (speedup 1.0000x reference)
import jax
import jax.numpy as jnp
from jax.experimental import pallas as pl


def kernel(features, edge_index, W1, b1, W2, b2):
    raise NotImplementedError("write your pallas kernel here")


# speedup vs baseline: 5.3479x; 5.3479x over previous
import functools

import jax
import jax.numpy as jnp
from jax import lax
from jax.experimental import pallas as pl
from jax.experimental.pallas import tpu as pltpu
from jax.experimental.pallas import tpu_sc as plsc

N = 10000
NP = 10240
E = 160000
D = 300
DP = 320
NCH = 4
HC = DP // NCH
CPS = NCH // 2
NT = 16
EPT = E // NT
K = 128
NB = (EPT + K - 1) // K
EPAD = NB * K - EPT
RPT = NP // NT
BM = 640
BMF = 400


def _mm_first_body(x_ref, w_ref, o_ref):
    y = jnp.dot(x_ref[...], w_ref[...], preferred_element_type=jnp.float32)
    for j in range(NCH):
        o_ref[j] = y[:, j * HC:(j + 1) * HC]


def _mm_mid_body(a_ref, b_ref, w_ref, o_ref):
    h = jnp.concatenate([a_ref[j] for j in range(NCH)], axis=-1)
    h = jnp.maximum(h + b_ref[...], 0.0)
    y = jnp.dot(h, w_ref[...], preferred_element_type=jnp.float32)
    for j in range(NCH):
        o_ref[j] = y[:, j * HC:(j + 1) * HC]


def _final_body(a_ref, b_ref, o_ref):
    h = jnp.concatenate([a_ref[j] for j in range(NCH)], axis=-1)
    o_ref[...] = jnp.maximum(h + b_ref[...], 0.0)[:, :D]


def _mm_first(x, w):
    return pl.pallas_call(
        _mm_first_body,
        grid=(NP // BM,),
        in_specs=[
            pl.BlockSpec((BM, D), lambda i: (i, 0)),
            pl.BlockSpec((D, DP), lambda i: (0, 0)),
        ],
        out_specs=pl.BlockSpec((NCH, BM, HC), lambda i: (0, i, 0)),
        out_shape=jax.ShapeDtypeStruct((NCH, NP, HC), jnp.float32),
    )(x, w)


def _mm_mid(a, b, w):
    return pl.pallas_call(
        _mm_mid_body,
        grid=(NP // BM,),
        in_specs=[
            pl.BlockSpec((NCH, BM, HC), lambda i: (0, i, 0)),
            pl.BlockSpec((1, DP), lambda i: (0, 0)),
            pl.BlockSpec((DP, DP), lambda i: (0, 0)),
        ],
        out_specs=pl.BlockSpec((NCH, BM, HC), lambda i: (0, i, 0)),
        out_shape=jax.ShapeDtypeStruct((NCH, NP, HC), jnp.float32),
    )(a, b, w)


def _final(a, b):
    return pl.pallas_call(
        _final_body,
        grid=(N // BMF,),
        in_specs=[
            pl.BlockSpec((NCH, BMF, HC), lambda i: (0, i, 0)),
            pl.BlockSpec((1, DP), lambda i: (0, 0)),
        ],
        out_specs=pl.BlockSpec((BMF, D), lambda i: (i, 0)),
        out_shape=jax.ShapeDtypeStruct((N, D), jnp.float32),
    )(a, b)


def _agg_body(table_hbm, src_hbm, dst_hbm, out_hbm,
              src_v, dst_v, stage_v, acc_sh, sem):
    c = lax.axis_index("c")
    t = lax.axis_index("s")

    pltpu.sync_copy(src_hbm.at[t], src_v)
    pltpu.sync_copy(dst_hbm.at[t], dst_v)

    for kk in range(CPS):
        tbl = table_hbm.at[c * CPS + kk]

        for chunk in range(RPT // K):
            off = t * RPT + chunk * K
            pltpu.sync_copy(tbl.at[pl.ds(off, K)], stage_v.at[0])
            pltpu.sync_copy(stage_v.at[0], acc_sh.at[pl.ds(off, K)])
        plsc.subcore_barrier()

        def body(b, carry):
            pltpu.async_copy(tbl.at[src_v.at[b]], stage_v.at[0], sem).wait()
            pltpu.sync_copy(stage_v.at[0], acc_sh.at[dst_v.at[b]], add=True)
            return carry

        lax.fori_loop(0, NB, body, 0)
        plsc.subcore_barrier()

        pltpu.sync_copy(acc_sh.at[pl.ds(t * RPT, RPT)],
                        out_hbm.at[c * CPS + kk].at[pl.ds(t * RPT, RPT)])


_agg = functools.partial(
    pl.kernel,
    out_type=jax.ShapeDtypeStruct((NCH, NP, HC), jnp.float32),
    mesh=plsc.VectorSubcoreMesh(core_axis_name="c", subcore_axis_name="s"),
    scratch_types=[
        pltpu.VMEM((NB, K), jnp.int32),
        pltpu.VMEM((NB, K), jnp.int32),
        pltpu.VMEM((2, K, HC), jnp.float32),
        pltpu.VMEM_SHARED((NP, HC), jnp.float32),
        pltpu.SemaphoreType.DMA,
    ],
    compiler_params=pltpu.CompilerParams(use_tc_tiling_on_sc=False),
)(_agg_body)


def kernel(features, edge_index, W1, b1, W2, b2):
    src = edge_index[0].astype(jnp.int32).reshape(NT, EPT)
    dst = edge_index[1].astype(jnp.int32).reshape(NT, EPT)
    pad_src = jnp.broadcast_to(
        jnp.arange(EPAD, dtype=jnp.int32)[None, :], (NT, EPAD))
    pad_dst = jnp.broadcast_to(
        (N + (jnp.arange(EPAD, dtype=jnp.int32) % (NP - N)))[None, :],
        (NT, EPAD))
    srcp = jnp.concatenate([src, pad_src], axis=1).reshape(NT, NB, K)
    dstp = jnp.concatenate([dst, pad_dst], axis=1).reshape(NT, NB, K)

    xp = jnp.pad(features, ((0, NP - N), (0, 0)))
    w1p = jnp.pad(W1.T, ((0, 0), (0, DP - D)))
    b1p = jnp.pad(b1, (0, DP - D)).reshape(1, DP)
    w2p = jnp.pad(W2.T, ((0, DP - D), (0, DP - D)))
    b2p = jnp.pad(b2, (0, DP - D)).reshape(1, DP)

    xw = _mm_first(xp, w1p)
    ag1 = _agg(xw, srcp, dstp)
    xw2 = _mm_mid(ag1, b1p, w2p)
    ag2 = _agg(xw2, srcp, dstp)
    return _final(ag2, b2p)


# speedup vs baseline: 7.3894x; 1.3817x over previous
import functools

import jax
import jax.numpy as jnp
from jax import lax
from jax.experimental import pallas as pl
from jax.experimental.pallas import tpu as pltpu
from jax.experimental.pallas import tpu_sc as plsc

N = 10000
NP = 10240
E = 160000
D = 300
DP = 320
NCH = 4
HC = DP // NCH
CPS = NCH // 2
NT = 16
EPT = E // NT
K = 128
NB = (EPT + K - 1) // K
EPAD = NB * K - EPT
RPT = NP // NT
BM = 640
BMF = 400


def _mm_first_body(x_ref, w_ref, o_ref):
    y = jnp.dot(x_ref[...], w_ref[...], preferred_element_type=jnp.float32)
    for j in range(NCH):
        o_ref[j] = y[:, j * HC:(j + 1) * HC]


def _mm_mid_body(a_ref, b_ref, w_ref, o_ref):
    h = jnp.concatenate([a_ref[j] for j in range(NCH)], axis=-1)
    h = jnp.maximum(h + b_ref[...], 0.0)
    y = jnp.dot(h, w_ref[...], preferred_element_type=jnp.float32)
    for j in range(NCH):
        o_ref[j] = y[:, j * HC:(j + 1) * HC]


def _final_body(a_ref, b_ref, o_ref):
    h = jnp.concatenate([a_ref[j] for j in range(NCH)], axis=-1)
    o_ref[...] = jnp.maximum(h + b_ref[...], 0.0)[:, :D]


def _mm_first(x, w):
    return pl.pallas_call(
        _mm_first_body,
        grid=(NP // BM,),
        in_specs=[
            pl.BlockSpec((BM, D), lambda i: (i, 0)),
            pl.BlockSpec((D, DP), lambda i: (0, 0)),
        ],
        out_specs=pl.BlockSpec((NCH, BM, HC), lambda i: (0, i, 0)),
        out_shape=jax.ShapeDtypeStruct((NCH, NP, HC), jnp.float32),
    )(x, w)


def _mm_mid(a, b, w):
    return pl.pallas_call(
        _mm_mid_body,
        grid=(NP // BM,),
        in_specs=[
            pl.BlockSpec((NCH, BM, HC), lambda i: (0, i, 0)),
            pl.BlockSpec((1, DP), lambda i: (0, 0)),
            pl.BlockSpec((DP, DP), lambda i: (0, 0)),
        ],
        out_specs=pl.BlockSpec((NCH, BM, HC), lambda i: (0, i, 0)),
        out_shape=jax.ShapeDtypeStruct((NCH, NP, HC), jnp.float32),
    )(a, b, w)


def _final(a, b):
    return pl.pallas_call(
        _final_body,
        grid=(N // BMF,),
        in_specs=[
            pl.BlockSpec((NCH, BMF, HC), lambda i: (0, i, 0)),
            pl.BlockSpec((1, DP), lambda i: (0, 0)),
        ],
        out_specs=pl.BlockSpec((BMF, D), lambda i: (i, 0)),
        out_shape=jax.ShapeDtypeStruct((N, D), jnp.float32),
    )(a, b)


def _agg_body(table_hbm, src_hbm, dst_hbm, out_hbm,
              src_v, dst_v, stage_v, acc_sh, sem_a, sem_b):
    c = lax.axis_index("c")
    t = lax.axis_index("s")

    pltpu.sync_copy(src_hbm.at[t], src_v)
    pltpu.sync_copy(dst_hbm.at[t], dst_v)

    for kk in range(CPS):
        tbl = table_hbm.at[c * CPS + kk]

        for chunk in range(RPT // K):
            off = t * RPT + chunk * K
            pltpu.sync_copy(tbl.at[pl.ds(off, K)], stage_v.at[0])
            pltpu.sync_copy(stage_v.at[0], acc_sh.at[pl.ds(off, K)])
        plsc.subcore_barrier()

        def g_start(b, slot, sem):
            pltpu.async_copy(tbl.at[src_v.at[b]], stage_v.at[slot], sem)

        def g_wait(slot, sem):
            pltpu.make_async_copy(
                tbl.at[src_v.at[0]], stage_v.at[slot], sem).wait()

        def scat(b, slot):
            pltpu.sync_copy(stage_v.at[slot], acc_sh.at[dst_v.at[b]],
                            add=True)

        g_start(0, 0, sem_a)

        def body(i, carry):
            b = 2 * i
            g_start(b + 1, 1, sem_b)
            g_wait(0, sem_a)
            scat(b, 0)
            g_start(b + 2, 0, sem_a)
            g_wait(1, sem_b)
            scat(b + 1, 1)
            return carry

        lax.fori_loop(0, (NB - 1) // 2, body, 0)
        g_wait(0, sem_a)
        scat(NB - 1, 0)
        plsc.subcore_barrier()

        pltpu.sync_copy(acc_sh.at[pl.ds(t * RPT, RPT)],
                        out_hbm.at[c * CPS + kk].at[pl.ds(t * RPT, RPT)])


_agg = functools.partial(
    pl.kernel,
    out_type=jax.ShapeDtypeStruct((NCH, NP, HC), jnp.float32),
    mesh=plsc.VectorSubcoreMesh(core_axis_name="c", subcore_axis_name="s"),
    scratch_types=[
        pltpu.VMEM((NB, K), jnp.int32),
        pltpu.VMEM((NB, K), jnp.int32),
        pltpu.VMEM((2, K, HC), jnp.float32),
        pltpu.VMEM_SHARED((NP, HC), jnp.float32),
        pltpu.SemaphoreType.DMA,
        pltpu.SemaphoreType.DMA,
    ],
    compiler_params=pltpu.CompilerParams(use_tc_tiling_on_sc=False),
)(_agg_body)


def kernel(features, edge_index, W1, b1, W2, b2):
    src = edge_index[0].astype(jnp.int32).reshape(NT, EPT)
    dst = edge_index[1].astype(jnp.int32).reshape(NT, EPT)
    pad_src = jnp.broadcast_to(
        jnp.arange(EPAD, dtype=jnp.int32)[None, :], (NT, EPAD))
    pad_dst = jnp.broadcast_to(
        (N + (jnp.arange(EPAD, dtype=jnp.int32) % (NP - N)))[None, :],
        (NT, EPAD))
    srcp = jnp.concatenate([src, pad_src], axis=1).reshape(NT, NB, K)
    dstp = jnp.concatenate([dst, pad_dst], axis=1).reshape(NT, NB, K)

    xp = jnp.pad(features, ((0, NP - N), (0, 0)))
    w1p = jnp.pad(W1.T, ((0, 0), (0, DP - D)))
    b1p = jnp.pad(b1, (0, DP - D)).reshape(1, DP)
    w2p = jnp.pad(W2.T, ((0, DP - D), (0, DP - D)))
    b2p = jnp.pad(b2, (0, DP - D)).reshape(1, DP)

    xw = _mm_first(xp, w1p)
    ag1 = _agg(xw, srcp, dstp)
    xw2 = _mm_mid(ag1, b1p, w2p)
    ag2 = _agg(xw2, srcp, dstp)
    return _final(ag2, b2p)


# speedup vs baseline: 8.2694x; 1.1191x over previous
import functools

import jax
import jax.numpy as jnp
from jax import lax
from jax.experimental import pallas as pl
from jax.experimental.pallas import tpu as pltpu
from jax.experimental.pallas import tpu_sc as plsc

N = 10000
NP = 10240
E = 160000
D = 300
DP = 320
NCH = 4
HC = DP // NCH
CPS = NCH // 2
NT = 16
EPT = E // NT
K = 128
NB = 80
EPAD = NB * K - EPT
NSLOT = 4
RPT = NP // NT
BM = 640
BMF = 400


def _mm_first_body(x_ref, w_ref, o_ref):
    y = jnp.dot(x_ref[...], w_ref[...], preferred_element_type=jnp.float32)
    for j in range(NCH):
        o_ref[j] = y[:, j * HC:(j + 1) * HC]


def _mm_mid_body(a_ref, b_ref, w_ref, o_ref):
    h = jnp.concatenate([a_ref[j] for j in range(NCH)], axis=-1)
    h = jnp.maximum(h + b_ref[...], 0.0)
    y = jnp.dot(h, w_ref[...], preferred_element_type=jnp.float32)
    for j in range(NCH):
        o_ref[j] = y[:, j * HC:(j + 1) * HC]


def _final_body(a_ref, b_ref, o_ref):
    h = jnp.concatenate([a_ref[j] for j in range(NCH)], axis=-1)
    o_ref[...] = jnp.maximum(h + b_ref[...], 0.0)[:, :D]


def _mm_first(x, w):
    return pl.pallas_call(
        _mm_first_body,
        grid=(N // BMF,),
        in_specs=[
            pl.BlockSpec((BMF, D), lambda i: (i, 0)),
            pl.BlockSpec((D, DP), lambda i: (0, 0)),
        ],
        out_specs=pl.BlockSpec((NCH, BMF, HC), lambda i: (0, i, 0)),
        out_shape=jax.ShapeDtypeStruct((NCH, NP, HC), jnp.float32),
    )(x, w)


def _mm_mid(a, b, w):
    return pl.pallas_call(
        _mm_mid_body,
        grid=(NP // BM,),
        in_specs=[
            pl.BlockSpec((NCH, BM, HC), lambda i: (0, i, 0)),
            pl.BlockSpec((1, DP), lambda i: (0, 0)),
            pl.BlockSpec((DP, DP), lambda i: (0, 0)),
        ],
        out_specs=pl.BlockSpec((NCH, BM, HC), lambda i: (0, i, 0)),
        out_shape=jax.ShapeDtypeStruct((NCH, NP, HC), jnp.float32),
    )(a, b, w)


def _final(a, b):
    return pl.pallas_call(
        _final_body,
        grid=(N // BMF,),
        in_specs=[
            pl.BlockSpec((NCH, BMF, HC), lambda i: (0, i, 0)),
            pl.BlockSpec((1, DP), lambda i: (0, 0)),
        ],
        out_specs=pl.BlockSpec((BMF, D), lambda i: (i, 0)),
        out_shape=jax.ShapeDtypeStruct((N, D), jnp.float32),
    )(a, b)


def _agg_body(table_hbm, src_hbm, dst_hbm, out_hbm,
              src_v, dst_v, stage_v, acc_sh, *sems):
    gsem = sems[:NSLOT]
    ssem = sems[NSLOT:]
    c = lax.axis_index("c")
    t = lax.axis_index("s")

    pltpu.sync_copy(src_hbm.at[t], src_v)
    pltpu.sync_copy(dst_hbm.at[t], dst_v)

    for kk in range(CPS):
        tbl = table_hbm.at[c * CPS + kk]

        for chunk in range(RPT // K):
            off = t * RPT + chunk * K
            pltpu.sync_copy(tbl.at[pl.ds(off, K)], stage_v.at[0])
            pltpu.sync_copy(stage_v.at[0], acc_sh.at[pl.ds(off, K)])
        plsc.subcore_barrier()

        def g_start(b, s):
            pltpu.async_copy(tbl.at[src_v.at[b]], stage_v.at[s], gsem[s])

        def g_wait(s):
            pltpu.make_async_copy(
                tbl.at[src_v.at[0]], stage_v.at[s], gsem[s]).wait()

        def s_start(b, s):
            pltpu.async_copy(stage_v.at[s], acc_sh.at[dst_v.at[b]],
                             ssem[s], add=True)

        def s_wait(s):
            pltpu.make_async_copy(
                stage_v.at[s], acc_sh.at[dst_v.at[0]], ssem[s]).wait()

        for s in range(NSLOT):
            g_start(s, s)

        def body(i, carry):
            b = i * NSLOT
            for s in range(NSLOT):
                g_wait(s)
                s_start(b + s, s)

            @pl.when(b + NSLOT < NB)
            def _():
                for s in range(NSLOT):
                    s_wait(s)
                    g_start(b + NSLOT + s, s)

            return carry

        lax.fori_loop(0, NB // NSLOT, body, 0)
        for s in range(NSLOT):
            s_wait(s)
        plsc.subcore_barrier()

        pltpu.sync_copy(acc_sh.at[pl.ds(t * RPT, RPT)],
                        out_hbm.at[c * CPS + kk].at[pl.ds(t * RPT, RPT)])


_agg = functools.partial(
    pl.kernel,
    out_type=jax.ShapeDtypeStruct((NCH, NP, HC), jnp.float32),
    mesh=plsc.VectorSubcoreMesh(core_axis_name="c", subcore_axis_name="s"),
    scratch_types=[
        pltpu.VMEM((NB, K), jnp.int32),
        pltpu.VMEM((NB, K), jnp.int32),
        pltpu.VMEM((NSLOT, K, HC), jnp.float32),
        pltpu.VMEM_SHARED((NP, HC), jnp.float32),
    ] + [pltpu.SemaphoreType.DMA] * (2 * NSLOT),
    compiler_params=pltpu.CompilerParams(use_tc_tiling_on_sc=False),
)(_agg_body)


def kernel(features, edge_index, W1, b1, W2, b2):
    src = edge_index[0].astype(jnp.int32).reshape(NT, EPT)
    dst = edge_index[1].astype(jnp.int32).reshape(NT, EPT)
    pad_src = jnp.broadcast_to(
        jnp.arange(EPAD, dtype=jnp.int32)[None, :], (NT, EPAD))
    pad_dst = jnp.broadcast_to(
        (N + (jnp.arange(EPAD, dtype=jnp.int32) % (NP - N)))[None, :],
        (NT, EPAD))
    srcp = jnp.concatenate([src, pad_src], axis=1).reshape(NT, NB, K)
    dstp = jnp.concatenate([dst, pad_dst], axis=1).reshape(NT, NB, K)

    w1p = jnp.pad(W1.T, ((0, 0), (0, DP - D)))
    b1p = jnp.pad(b1, (0, DP - D)).reshape(1, DP)
    w2p = jnp.pad(W2.T, ((0, DP - D), (0, DP - D)))
    b2p = jnp.pad(b2, (0, DP - D)).reshape(1, DP)

    xw = _mm_first(features, w1p)
    ag1 = _agg(xw, srcp, dstp)
    xw2 = _mm_mid(ag1, b1p, w2p)
    ag2 = _agg(xw2, srcp, dstp)
    return _final(ag2, b2p)


# speedup vs baseline: 8.6028x; 1.0403x over previous
import functools

import jax
import jax.numpy as jnp
from jax import lax
from jax.experimental import pallas as pl
from jax.experimental.pallas import tpu as pltpu
from jax.experimental.pallas import tpu_sc as plsc

N = 10000
NP = 10240
E = 160000
D = 300
DP = 320
NCH = 4
HC = DP // NCH
NT = 16
EPT = E // NT
K = 128
NB = 80
EPAD = NB * K - EPT
NSLOT = 4
RPT = NP // NT
BM = 640
BMF = 400


def _mm_first_body(x_ref, w_ref, o_ref):
    y = jnp.dot(x_ref[...], w_ref[...], preferred_element_type=jnp.float32)
    o_ref[0] = y[:, :HC]
    o_ref[1] = y[:, HC:]


def _mm_midA_body(a_ref, b_ref, w_ref, o_ref):
    h = jnp.concatenate(
        [jnp.maximum(a_ref[0] + b_ref[0, 0:HC], 0.0),
         jnp.maximum(a_ref[1] + b_ref[0, 2 * HC:3 * HC], 0.0)], axis=-1)
    o_ref[...] = jnp.dot(h, w_ref[...], preferred_element_type=jnp.float32)


def _mm_midB_body(a_ref, p_ref, b_ref, w_ref, oA_ref, oB_ref):
    h = jnp.concatenate(
        [jnp.maximum(a_ref[0] + b_ref[0, HC:2 * HC], 0.0),
         jnp.maximum(a_ref[1] + b_ref[0, 3 * HC:], 0.0)], axis=-1)
    y = p_ref[...] + jnp.dot(h, w_ref[...],
                             preferred_element_type=jnp.float32)
    oA_ref[0] = y[:, 0:HC]
    oA_ref[1] = y[:, 2 * HC:3 * HC]
    oB_ref[0] = y[:, HC:2 * HC]
    oB_ref[1] = y[:, 3 * HC:]


def _final_body(aA_ref, aB_ref, b_ref, o_ref):
    h = jnp.concatenate(
        [aA_ref[0], aB_ref[0], aA_ref[1], aB_ref[1]], axis=-1)
    o_ref[...] = jnp.maximum(h + b_ref[...], 0.0)[:, :D]


def _mm_first(x, w):
    return pl.pallas_call(
        _mm_first_body,
        grid=(N // BMF,),
        in_specs=[
            pl.BlockSpec((BMF, D), lambda i: (i, 0)),
            pl.BlockSpec((D, 2 * HC), lambda i: (0, 0)),
        ],
        out_specs=pl.BlockSpec((2, BMF, HC), lambda i: (0, i, 0)),
        out_shape=jax.ShapeDtypeStruct((2, NP, HC), jnp.float32),
    )(x, w)


def _mm_midA(a, b, w):
    return pl.pallas_call(
        _mm_midA_body,
        grid=(NP // BM,),
        in_specs=[
            pl.BlockSpec((2, BM, HC), lambda i: (0, i, 0)),
            pl.BlockSpec((1, DP), lambda i: (0, 0)),
            pl.BlockSpec((2 * HC, DP), lambda i: (0, 0)),
        ],
        out_specs=pl.BlockSpec((BM, DP), lambda i: (i, 0)),
        out_shape=jax.ShapeDtypeStruct((NP, DP), jnp.float32),
    )(a, b, w)


def _mm_midB(a, p, b, w):
    return pl.pallas_call(
        _mm_midB_body,
        grid=(NP // BM,),
        in_specs=[
            pl.BlockSpec((2, BM, HC), lambda i: (0, i, 0)),
            pl.BlockSpec((BM, DP), lambda i: (i, 0)),
            pl.BlockSpec((1, DP), lambda i: (0, 0)),
            pl.BlockSpec((2 * HC, DP), lambda i: (0, 0)),
        ],
        out_specs=[
            pl.BlockSpec((2, BM, HC), lambda i: (0, i, 0)),
            pl.BlockSpec((2, BM, HC), lambda i: (0, i, 0)),
        ],
        out_shape=[
            jax.ShapeDtypeStruct((2, NP, HC), jnp.float32),
            jax.ShapeDtypeStruct((2, NP, HC), jnp.float32),
        ],
    )(a, p, b, w)


def _final(aA, aB, b):
    return pl.pallas_call(
        _final_body,
        grid=(N // BMF,),
        in_specs=[
            pl.BlockSpec((2, BMF, HC), lambda i: (0, i, 0)),
            pl.BlockSpec((2, BMF, HC), lambda i: (0, i, 0)),
            pl.BlockSpec((1, DP), lambda i: (0, 0)),
        ],
        out_specs=pl.BlockSpec((BMF, D), lambda i: (i, 0)),
        out_shape=jax.ShapeDtypeStruct((N, D), jnp.float32),
    )(aA, aB, b)


def _agg_body(table_hbm, src_hbm, dst_hbm, out_hbm,
              src_v, dst_v, stage_v, acc_sh, *sems):
    gsem = sems[:NSLOT]
    ssem = sems[NSLOT:]
    c = lax.axis_index("c")
    t = lax.axis_index("s")
    tbl = table_hbm.at[c]

    pltpu.sync_copy(src_hbm.at[t], src_v)
    pltpu.sync_copy(dst_hbm.at[t], dst_v)

    nck = RPT // K
    for chunk in range(min(NSLOT, nck)):
        off = t * RPT + chunk * K
        pltpu.async_copy(tbl.at[pl.ds(off, K)], stage_v.at[chunk],
                         gsem[chunk])
    for chunk in range(nck):
        s = chunk % NSLOT
        off = t * RPT + chunk * K
        pltpu.make_async_copy(tbl.at[pl.ds(off, K)], stage_v.at[s],
                              gsem[s]).wait()
        pltpu.async_copy(stage_v.at[s], acc_sh.at[pl.ds(off, K)], ssem[s])
        nxt = chunk + NSLOT
        if nxt < nck:
            noff = t * RPT + nxt * K
            pltpu.make_async_copy(stage_v.at[s], acc_sh.at[pl.ds(off, K)],
                                  ssem[s]).wait()
            pltpu.async_copy(tbl.at[pl.ds(noff, K)], stage_v.at[s],
                             gsem[s])
    for chunk in range(max(0, nck - NSLOT), nck):
        s = chunk % NSLOT
        off = t * RPT + chunk * K
        pltpu.make_async_copy(stage_v.at[s], acc_sh.at[pl.ds(off, K)],
                              ssem[s]).wait()
    plsc.subcore_barrier()

    def g_start(b, s):
        pltpu.async_copy(tbl.at[src_v.at[b]], stage_v.at[s], gsem[s])

    def g_wait(s):
        pltpu.make_async_copy(
            tbl.at[src_v.at[0]], stage_v.at[s], gsem[s]).wait()

    def s_start(b, s):
        pltpu.async_copy(stage_v.at[s], acc_sh.at[dst_v.at[b]],
                         ssem[s], add=True)

    def s_wait(s):
        pltpu.make_async_copy(
            stage_v.at[s], acc_sh.at[dst_v.at[0]], ssem[s]).wait()

    for s in range(NSLOT):
        g_start(s, s)

    def body(i, carry):
        b = i * NSLOT
        for s in range(NSLOT):
            g_wait(s)
            s_start(b + s, s)

        @pl.when(b + NSLOT < NB)
        def _():
            for s in range(NSLOT):
                s_wait(s)
                g_start(b + NSLOT + s, s)

        return carry

    lax.fori_loop(0, NB // NSLOT, body, 0)
    for s in range(NSLOT):
        s_wait(s)
    plsc.subcore_barrier()

    pltpu.sync_copy(acc_sh.at[pl.ds(t * RPT, RPT)],
                    out_hbm.at[c].at[pl.ds(t * RPT, RPT)])


_agg = functools.partial(
    pl.kernel,
    out_type=jax.ShapeDtypeStruct((2, NP, HC), jnp.float32),
    mesh=plsc.VectorSubcoreMesh(core_axis_name="c", subcore_axis_name="s"),
    scratch_types=[
        pltpu.VMEM((NB, K), jnp.int32),
        pltpu.VMEM((NB, K), jnp.int32),
        pltpu.VMEM((NSLOT, K, HC), jnp.float32),
        pltpu.VMEM_SHARED((NP, HC), jnp.float32),
    ] + [pltpu.SemaphoreType.DMA] * (2 * NSLOT),
    compiler_params=pltpu.CompilerParams(use_tc_tiling_on_sc=False),
)(_agg_body)


def _cols(w, j0, j1):
    return jnp.concatenate(
        [w[:, j0 * HC:(j0 + 1) * HC], w[:, j1 * HC:(j1 + 1) * HC]], axis=1)


def _rows(w, j0, j1):
    return jnp.concatenate(
        [w[j0 * HC:(j0 + 1) * HC], w[j1 * HC:(j1 + 1) * HC]], axis=0)


def kernel(features, edge_index, W1, b1, W2, b2):
    src = edge_index[0].astype(jnp.int32).reshape(NT, EPT)
    dst = edge_index[1].astype(jnp.int32).reshape(NT, EPT)
    pad_src = jnp.broadcast_to(
        jnp.arange(EPAD, dtype=jnp.int32)[None, :], (NT, EPAD))
    pad_dst = jnp.broadcast_to(
        (N + (jnp.arange(EPAD, dtype=jnp.int32) % (NP - N)))[None, :],
        (NT, EPAD))
    srcp = jnp.concatenate([src, pad_src], axis=1).reshape(NT, NB, K)
    dstp = jnp.concatenate([dst, pad_dst], axis=1).reshape(NT, NB, K)

    w1p = jnp.pad(W1.T, ((0, 0), (0, DP - D)))
    b1p = jnp.pad(b1, (0, DP - D)).reshape(1, DP)
    w2p = jnp.pad(W2.T, ((0, DP - D), (0, DP - D)))
    b2p = jnp.pad(b2, (0, DP - D)).reshape(1, DP)

    xwA = _mm_first(features, _cols(w1p, 0, 2))
    xwB = _mm_first(features, _cols(w1p, 1, 3))
    agA = _agg(xwA, srcp, dstp)
    agB = _agg(xwB, srcp, dstp)

    yP = _mm_midA(agA, b1p, _rows(w2p, 0, 2))
    xw2A, xw2B = _mm_midB(agB, yP, b1p, _rows(w2p, 1, 3))
    ag2A = _agg(xw2A, srcp, dstp)
    ag2B = _agg(xw2B, srcp, dstp)
    return _final(ag2A, ag2B, b2p)
